# baseline (speedup 1.0000x reference)
import jax
import jax.numpy as jnp
from jax.experimental import pallas as pl

B = 16
T = 50
P = 8
H = 64
W = 64
N = B * T
LW = W * P
G = 40


def _body(xr, yr, dxr, dyr, oxr, oyr, out_ref):
    coli = (xr[...] / dxr[...] + oxr[...]).astype(jnp.int32)
    rowi = (yr[...] / dyr[...] + oyr[...]).astype(jnp.int32)
    lane = jax.lax.broadcasted_iota(jnp.int32, (G, LW), 1)
    wl = lane >> 3
    inb = (coli >= 0) & (coli < W) & (rowi >= 0) & (rowi < H)
    tgt = jnp.where(inb & (coli == wl), rowi, -1)
    hio = jax.lax.broadcasted_iota(jnp.int32, (G, H, LW), 1)
    out_ref[...] = (hio == tgt[:, None, :]).astype(jnp.float32)


def kernel(x, resolution, origin):
    pts = x.reshape(N, P, 2)
    xt = jnp.tile(pts[:, :, 0], (1, W))
    yt = jnp.tile(pts[:, :, 1], (1, W))
    res = resolution.reshape(N, 2)
    org = origin.reshape(N, 2)
    dxt = jnp.tile(res[:, 0:1], (1, LW))
    dyt = jnp.tile(res[:, 1:2], (1, LW))
    oxt = jnp.tile(org[:, 1:2], (1, LW))
    oyt = jnp.tile(org[:, 0:1], (1, LW))

    out = pl.pallas_call(
        _body,
        grid=(N // G,),
        in_specs=[pl.BlockSpec((G, LW), lambda i: (i, 0))] * 6,
        out_specs=pl.BlockSpec((G, H, LW), lambda i: (i, 0, 0)),
        out_shape=jax.ShapeDtypeStruct((N, H, LW), jnp.float32),
    )(xt, yt, dxt, dyt, oxt, oyt)
    return out.reshape(B, T, H, W, P)
